# Initial kernel scaffold; baseline (speedup 1.0000x reference)
import jax
import jax.numpy as jnp
from jax.experimental import pallas as pl


def kernel(x, edge_index0, edge_weight0, edge_index1, edge_weight1, weights_0, sdweight_0, sdweight_1):
    raise NotImplementedError("write your pallas kernel here")


# speedup vs baseline: 3.7913x; 3.7913x over previous
import functools

import jax
import jax.numpy as jnp
from jax import lax
from jax.experimental import pallas as pl
from jax.experimental.pallas import tpu as pltpu
from jax.experimental.pallas import tpu_sc as plsc

_N = 10000
_D = 128
_E = 320000
_NC = 2
_NS = 16
_LANES = 16
_CHUNK = 80


def _sc_spmm(x, src, dst, w, zeros):
    total = src.shape[0]
    per_tile = total // (_NC * _NS)
    n_chunks = per_tile // _CHUNK
    assert per_tile * _NC * _NS == total and n_chunks * _CHUNK == per_tile
    rows_a = 624
    rows_last = _N - (_NS - 1) * rows_a

    mesh = plsc.VectorSubcoreMesh(core_axis_name="c", subcore_axis_name="s")

    @functools.partial(
        pl.kernel,
        mesh=mesh,
        out_type=jax.ShapeDtypeStruct((_NC, _N, _D), jnp.float32),
        scratch_types=[
            pltpu.VMEM((_CHUNK,), jnp.int32),
            pltpu.VMEM((_CHUNK,), jnp.int32),
            pltpu.VMEM((_CHUNK,), jnp.float32),
            pltpu.VMEM((_CHUNK, _D), jnp.float32),
            pltpu.VMEM_SHARED((_N, _D), jnp.float32),
            pltpu.SemaphoreType.DMA,
        ],
    )
    def spmm(x_hbm, src_hbm, dst_hbm, w_hbm, zeros_hbm, out_hbm,
             src_v, dst_v, w_v, rows_v, acc, sem):
        c = lax.axis_index("c")
        s = lax.axis_index("s")
        row0 = s * rows_a
        @pl.when(s < _NS - 1)
        def _():
            pltpu.sync_copy(zeros_hbm.at[pl.ds(0, rows_a)],
                            acc.at[pl.ds(row0, rows_a)])

        @pl.when(s == _NS - 1)
        def _():
            pltpu.sync_copy(zeros_hbm,
                            acc.at[pl.ds((_NS - 1) * rows_a, rows_last)])

        plsc.subcore_barrier()

        tile_base = (c * _NS + s) * per_tile

        def chunk_body(it, carry):
            base = tile_base + it * _CHUNK
            pltpu.sync_copy(src_hbm.at[pl.ds(base, _CHUNK)], src_v)
            pltpu.sync_copy(dst_hbm.at[pl.ds(base, _CHUNK)], dst_v)
            pltpu.sync_copy(w_hbm.at[pl.ds(base, _CHUNK)], w_v)
            pltpu.async_copy(x_hbm.at[src_v], rows_v, sem).wait()

            def scale_grp(g, carry2):
                wv = w_v[pl.ds(g * _LANES, _LANES)]
                for j in range(_LANES):
                    e = g * _LANES + j
                    ws = wv[j]
                    for k in range(_D // _LANES):
                        sl = pl.ds(k * _LANES, _LANES)
                        rows_v[e, sl] = rows_v[e, sl] * ws
                return carry2

            lax.fori_loop(0, _CHUNK // _LANES, scale_grp, 0, unroll=False)
            pltpu.sync_copy(rows_v, acc.at[dst_v], add=True)
            return carry

        lax.fori_loop(0, n_chunks, chunk_body, 0, unroll=False)
        plsc.subcore_barrier()

        @pl.when(s < _NS - 1)
        def _():
            pltpu.sync_copy(acc.at[pl.ds(row0, rows_a)],
                            out_hbm.at[c, pl.ds(row0, rows_a)])

        @pl.when(s == _NS - 1)
        def _():
            pltpu.sync_copy(acc.at[pl.ds((_NS - 1) * rows_a, rows_last)],
                            out_hbm.at[c, pl.ds((_NS - 1) * rows_a, rows_last)])

    return spmm(x, src, dst, w, zeros)


def _tc_combine(p, sda, sdb, wmat):
    blk = 1000

    def body(p0_ref, p1_ref, sda_ref, sdb_ref, w_ref, o_ref):
        acc = p0_ref[0] * sda_ref[...] + p1_ref[0] * sdb_ref[...]
        y = jnp.dot(acc, w_ref[...], preferred_element_type=jnp.float32)
        o_ref[...] = jnp.maximum(y, 0.0)

    return pl.pallas_call(
        body,
        grid=(_N // blk,),
        in_specs=[
            pl.BlockSpec((1, blk, _D), lambda i: (0, i, 0)),
            pl.BlockSpec((1, blk, _D), lambda i: (1, i, 0)),
            pl.BlockSpec((1, _D), lambda i: (0, 0)),
            pl.BlockSpec((1, _D), lambda i: (0, 0)),
            pl.BlockSpec((_D, _D), lambda i: (0, 0)),
        ],
        out_specs=pl.BlockSpec((blk, _D), lambda i: (i, 0)),
        out_shape=jax.ShapeDtypeStruct((_N, _D), jnp.float32),
    )(p, p, sda.reshape(1, _D), sdb.reshape(1, _D), wmat)


def kernel(x, edge_index0, edge_weight0, edge_index1, edge_weight1,
           weights_0, sdweight_0, sdweight_1):
    src = jnp.concatenate([edge_index0[1], edge_index1[1]])
    dst = jnp.concatenate([edge_index0[0], edge_index1[0]])
    w = jnp.concatenate([edge_weight0, edge_weight1])
    zeros = jnp.zeros((_N - (_NS - 1) * 624, _D), jnp.float32)
    p = _sc_spmm(x, src, dst, w, zeros)
    return _tc_combine(p, sdweight_0, sdweight_1, weights_0)
